# baseline (speedup 1.0000x reference)
import functools

import jax
import jax.numpy as jnp
from jax import lax
from jax.experimental import pallas as pl
from jax.experimental.pallas import tpu as pltpu
from jax.experimental.pallas import tpu_sc as plsc

_NC = 2
_NS = 16
_NW = _NC * _NS
_KC = 100

_PERM = (list(range(0, 16)) + list(range(32, 48))
         + list(range(16, 32)) + list(range(48, 64)))


def _transpose_body(e_ref, out_ref):
    d = e_ref.shape[0]
    u = jax.lax.bitcast_convert_type(e_ref[...].T, jnp.uint32)
    rb = (u + 0x7FFF + ((u >> 16) & 1)) >> 16
    w = rb[:, : d // 2] | (rb[:, d // 2:] << 16)
    n4 = w.shape[0] // 4
    out_ref[...] = jax.lax.bitcast_convert_type(
        jnp.concatenate([w[g * n4:(g + 1) * n4] for g in range(4)], axis=1),
        jnp.int32)


def _transpose(E, vb):
    D, V = E.shape
    nblk = pl.cdiv(V, vb)
    return pl.pallas_call(
        _transpose_body,
        grid=(nblk,),
        in_specs=[pl.BlockSpec((D, vb), lambda i: (0, i))],
        out_specs=pl.BlockSpec((vb // 4, 2 * D), lambda i: (i, 0)),
        out_shape=jax.ShapeDtypeStruct((nblk * vb // 4, 2 * D), jnp.int32),
    )(E)


def _make_pool(B, L, V, D):
    rpw = B * L // _NW
    ndw = B // _NW
    nch = rpw // _KC
    nbuf = 8
    mesh = plsc.VectorSubcoreMesh(core_axis_name="c", subcore_axis_name="s")

    @functools.partial(
        pl.kernel,
        out_type=jax.ShapeDtypeStruct((B, D), jnp.float32),
        mesh=mesh,
        scratch_types=(
            [pltpu.VMEM((nch, _KC), jnp.int32)]
            + [pltpu.VMEM((_KC, D // 2), jnp.int32)] * nbuf
            + [pltpu.VMEM((ndw, D), jnp.float32)]
            + [pltpu.SemaphoreType.DMA] * nbuf
        ),
        compiler_params=pltpu.CompilerParams(use_tc_tiling_on_sc=False,
                                             needs_layout_passes=False),
    )
    def pool(et, docs3, out, si,
             r0, r1, r2, r3, r4, r5, r6, r7, acc,
             s0, s1, s2, s3, s4, s5, s6, s7):
        rs = [r0, r1, r2, r3, r4, r5, r6, r7]
        ss = [s0, s1, s2, s3, s4, s5, s6, s7]
        sid = lax.axis_index("s")
        wid = sid * _NC + lax.axis_index("c")
        pltpu.sync_copy(docs3.at[wid], si)
        for j in range(nbuf - 1):
            pltpu.async_copy(et.at[si.at[j]], rs[j], ss[j])

        @pl.loop(0, nch, step=nbuf)
        def _(k):
            for j in range(nbuf):
                kk = k + j
                jn = (j + nbuf - 1) % nbuf
                pltpu.make_async_copy(et.at[si.at[kk]], rs[j], ss[j]).wait()
                rbuf = rs[j]

                def row_sum(r, c):
                    a0, a1, a2, a3 = c
                    hi_mask = jnp.full((16,), -65536, jnp.int32)
                    w0 = rbuf[r, pl.ds(0, 16)]
                    w1 = rbuf[r, pl.ds(16, 16)]
                    a0 += plsc.bitcast(w0 << 16, jnp.float32)
                    a1 += plsc.bitcast(w0 & hi_mask, jnp.float32)
                    a2 += plsc.bitcast(w1 << 16, jnp.float32)
                    a3 += plsc.bitcast(w1 & hi_mask, jnp.float32)
                    return a0, a1, a2, a3

                z = jnp.zeros((16,), jnp.float32)
                a0, a1, a2, a3 = lax.fori_loop(0, _KC, row_sum, (z, z, z, z),
                                               unroll=4)
                doc = kk // 2

                @pl.when(kk % 2 == 0)
                def _():
                    acc[doc, pl.ds(0, 16)] = a0
                    acc[doc, pl.ds(16, 16)] = a1
                    acc[doc, pl.ds(32, 16)] = a2
                    acc[doc, pl.ds(48, 16)] = a3

                @pl.when(kk % 2 == 1)
                def _():
                    plsc.addupdate(acc.at[doc, pl.ds(0, 16)], a0)
                    plsc.addupdate(acc.at[doc, pl.ds(16, 16)], a1)
                    plsc.addupdate(acc.at[doc, pl.ds(32, 16)], a2)
                    plsc.addupdate(acc.at[doc, pl.ds(48, 16)], a3)

                @pl.when(kk + nbuf - 1 < nch)
                def _():
                    pltpu.async_copy(et.at[si.at[kk + nbuf - 1]],
                                     rs[jn], ss[jn])

        pltpu.sync_copy(acc, out.at[pl.ds(wid * ndw, ndw)])

    return pool, rpw, nch


def _mlp_body(x_ref, dl_ref, w1_ref, b1_ref, w2_ref, b2_ref, w3_ref, b3_ref,
              o_ref):
    x = x_ref[...] / dl_ref[...]
    h = jnp.maximum(jnp.dot(x, w1_ref[...]) + b1_ref[...], 0.0)
    h = jnp.maximum(jnp.dot(h, w2_ref[...]) + b2_ref[...], 0.0)
    o_ref[...] = jnp.dot(h, w3_ref[...]) + b3_ref[...]


def _mlp(x, dl, W1, b1, W2, b2, W3, b3, bb):
    B, D = x.shape
    H = W1.shape[1]
    C = W3.shape[1]
    full = lambda s: pl.BlockSpec(s, lambda i: (0, 0))
    return pl.pallas_call(
        _mlp_body,
        grid=(B // bb,),
        in_specs=[
            pl.BlockSpec((bb, D), lambda i: (i, 0)),
            pl.BlockSpec((bb, 1), lambda i: (i, 0)),
            full((D, H)), full((1, H)),
            full((H, H)), full((1, H)),
            full((H, C)), full((1, C)),
        ],
        out_specs=pl.BlockSpec((bb, C), lambda i: (i, 0)),
        out_shape=jax.ShapeDtypeStruct((B, C), jnp.float32),
    )(x, dl, W1, b1.reshape(1, H), W2, b2.reshape(1, H), W3, b3.reshape(1, C))


def kernel(docs, embeddings_matrix, doc_lens, W1, b1, W2, b2, W3, b3):
    B, L = docs.shape
    D, V = embeddings_matrix.shape

    ETp = _transpose(embeddings_matrix, vb=2048)
    ET = ETp.reshape(ETp.shape[0] * 4, D // 2)

    docs_s = (((docs >> 11) << 11) + ((docs & 511) << 2) + ((docs >> 9) & 3))

    pool, rpw, nch = _make_pool(B, L, V, D)
    docs3 = docs_s.reshape(_NW, nch, _KC)
    sums = pool(ET, docs3)

    W1p = W1[jnp.array(_PERM), :]
    return _mlp(sums, doc_lens.reshape(B, 1), W1p, b1, W2, b2, W3, b3, bb=1024)


# speedup vs baseline: 1.0031x; 1.0031x over previous
import functools

import jax
import jax.numpy as jnp
from jax import lax
from jax.experimental import pallas as pl
from jax.experimental.pallas import tpu as pltpu
from jax.experimental.pallas import tpu_sc as plsc

_NC = 2
_NS = 16
_NW = _NC * _NS
_KC = 100

_PERM = (list(range(0, 16)) + list(range(32, 48))
         + list(range(16, 32)) + list(range(48, 64)))


def _transpose_body(e_ref, out_ref):
    d = e_ref.shape[0]
    u = jax.lax.bitcast_convert_type(e_ref[...].T, jnp.uint32)
    rb = (u + 0x7FFF + ((u >> 16) & 1)) >> 16
    w = rb[:, : d // 2] | (rb[:, d // 2:] << 16)
    n4 = w.shape[0] // 4
    out_ref[...] = jax.lax.bitcast_convert_type(
        jnp.concatenate([w[g * n4:(g + 1) * n4] for g in range(4)], axis=1),
        jnp.int32)


def _transpose(E, vb):
    D, V = E.shape
    nblk = pl.cdiv(V, vb)
    return pl.pallas_call(
        _transpose_body,
        grid=(nblk,),
        in_specs=[pl.BlockSpec((D, vb), lambda i: (0, i))],
        out_specs=pl.BlockSpec((vb // 4, 2 * D), lambda i: (i, 0)),
        out_shape=jax.ShapeDtypeStruct((nblk * vb // 4, 2 * D), jnp.int32),
    )(E)


def _make_pool(B, L, V, D):
    rpw = B * L // _NW
    ndw = B // _NW
    nch = rpw // _KC
    nbuf = 8
    mesh = plsc.VectorSubcoreMesh(core_axis_name="c", subcore_axis_name="s")

    @functools.partial(
        pl.kernel,
        out_type=jax.ShapeDtypeStruct((B, D), jnp.float32),
        mesh=mesh,
        scratch_types=(
            [pltpu.VMEM((nch, _KC), jnp.int32)]
            + [pltpu.VMEM((_KC, D // 2), jnp.int32)] * nbuf
            + [pltpu.VMEM((ndw, D), jnp.float32)]
            + [pltpu.SemaphoreType.DMA] * nbuf
        ),
        compiler_params=pltpu.CompilerParams(use_tc_tiling_on_sc=False,
                                             needs_layout_passes=False),
    )
    def pool(et, docs3, out, si,
             r0, r1, r2, r3, r4, r5, r6, r7, acc,
             s0, s1, s2, s3, s4, s5, s6, s7):
        rs = [r0, r1, r2, r3, r4, r5, r6, r7]
        ss = [s0, s1, s2, s3, s4, s5, s6, s7]
        sid = lax.axis_index("s")
        wid = sid * _NC + lax.axis_index("c")
        pltpu.sync_copy(docs3.at[wid], si)
        for j in range(nbuf - 1):
            pltpu.async_copy(et.at[si.at[j]], rs[j], ss[j])

        @pl.loop(0, nch, step=nbuf)
        def _(k):
            for j in range(nbuf):
                kk = k + j
                jn = (j + nbuf - 1) % nbuf
                pltpu.make_async_copy(et.at[si.at[kk]], rs[j], ss[j]).wait()
                rbuf = rs[j]

                def row_sum(r, c):
                    a0, a1, a2, a3 = c
                    hi_mask = jnp.full((16,), -65536, jnp.int32)
                    w0 = rbuf[r, pl.ds(0, 16)]
                    w1 = rbuf[r, pl.ds(16, 16)]
                    a0 += plsc.bitcast(w0 << 16, jnp.float32)
                    a1 += plsc.bitcast(w0 & hi_mask, jnp.float32)
                    a2 += plsc.bitcast(w1 << 16, jnp.float32)
                    a3 += plsc.bitcast(w1 & hi_mask, jnp.float32)
                    return a0, a1, a2, a3

                z = jnp.zeros((16,), jnp.float32)
                a0, a1, a2, a3 = lax.fori_loop(0, _KC, row_sum, (z, z, z, z),
                                               unroll=10)
                doc = kk // 2

                @pl.when(kk % 2 == 0)
                def _():
                    acc[doc, pl.ds(0, 16)] = a0
                    acc[doc, pl.ds(16, 16)] = a1
                    acc[doc, pl.ds(32, 16)] = a2
                    acc[doc, pl.ds(48, 16)] = a3

                @pl.when(kk % 2 == 1)
                def _():
                    plsc.addupdate(acc.at[doc, pl.ds(0, 16)], a0)
                    plsc.addupdate(acc.at[doc, pl.ds(16, 16)], a1)
                    plsc.addupdate(acc.at[doc, pl.ds(32, 16)], a2)
                    plsc.addupdate(acc.at[doc, pl.ds(48, 16)], a3)

                @pl.when(kk + nbuf - 1 < nch)
                def _():
                    pltpu.async_copy(et.at[si.at[kk + nbuf - 1]],
                                     rs[jn], ss[jn])

        pltpu.sync_copy(acc, out.at[pl.ds(wid * ndw, ndw)])

    return pool, rpw, nch


def _mlp_body(x_ref, dl_ref, w1_ref, b1_ref, w2_ref, b2_ref, w3_ref, b3_ref,
              o_ref):
    x = x_ref[...] / dl_ref[...]
    h = jnp.maximum(jnp.dot(x, w1_ref[...]) + b1_ref[...], 0.0)
    h = jnp.maximum(jnp.dot(h, w2_ref[...]) + b2_ref[...], 0.0)
    o_ref[...] = jnp.dot(h, w3_ref[...]) + b3_ref[...]


def _mlp(x, dl, W1, b1, W2, b2, W3, b3, bb):
    B, D = x.shape
    H = W1.shape[1]
    C = W3.shape[1]
    full = lambda s: pl.BlockSpec(s, lambda i: (0, 0))
    return pl.pallas_call(
        _mlp_body,
        grid=(B // bb,),
        in_specs=[
            pl.BlockSpec((bb, D), lambda i: (i, 0)),
            pl.BlockSpec((bb, 1), lambda i: (i, 0)),
            full((D, H)), full((1, H)),
            full((H, H)), full((1, H)),
            full((H, C)), full((1, C)),
        ],
        out_specs=pl.BlockSpec((bb, C), lambda i: (i, 0)),
        out_shape=jax.ShapeDtypeStruct((B, C), jnp.float32),
    )(x, dl, W1, b1.reshape(1, H), W2, b2.reshape(1, H), W3, b3.reshape(1, C))


def kernel(docs, embeddings_matrix, doc_lens, W1, b1, W2, b2, W3, b3):
    B, L = docs.shape
    D, V = embeddings_matrix.shape

    ETp = _transpose(embeddings_matrix, vb=2048)
    ET = ETp.reshape(ETp.shape[0] * 4, D // 2)

    docs_s = (((docs >> 11) << 11) + ((docs & 511) << 2) + ((docs >> 9) & 3))

    pool, rpw, nch = _make_pool(B, L, V, D)
    docs3 = docs_s.reshape(_NW, nch, _KC)
    sums = pool(ET, docs3)

    W1p = W1[jnp.array(_PERM), :]
    return _mlp(sums, doc_lens.reshape(B, 1), W1p, b1, W2, b2, W3, b3, bb=2048)
